# baseline (speedup 1.0000x reference)
import functools

import jax
import jax.numpy as jnp
import numpy as np
from jax import lax
from jax.experimental import pallas as pl
from jax.experimental.pallas import tpu as pltpu
from jax.experimental.pallas import tpu_sc as plsc

D = 128
DM = 256
H = 512
BQ = 2048
BS = 256
NB = 64
ROWS = 2 * BQ + 2 * BS

NC = 2
NS = 16
NW = NC * NS
RPW = ROWS // NW
IDX_ROWS = RPW * NB // 128
SELF_MINOR = RPW // 2


def _bf16_bits(x):
    h = lax.bitcast_convert_type(x.astype(jnp.bfloat16), jnp.int16)
    return jnp.bitwise_and(h.astype(jnp.int32), jnp.int32(0xFFFF))


def _bf16_bits(x):
    h = lax.bitcast_convert_type(x.astype(jnp.bfloat16), jnp.int16)
    return jnp.bitwise_and(h.astype(jnp.int32), jnp.int32(0xFFFF))


def _tables_body(emb, w, bias, ptab):
    e = emb[...].astype(jnp.bfloat16)
    w_all = w[...].astype(jnp.bfloat16)
    zr = lax.dot_general(e, w_all[:, :D], (((1,), (1,)), ((), ())),
                         preferred_element_type=jnp.float32)
    ze = lax.dot_general(e, w_all[:, D:], (((1,), (1,)), ((), ())),
                         preferred_element_type=jnp.float32) + bias[...]

    ptab[...] = jnp.bitwise_or(_bf16_bits(zr),
                               lax.shift_left(_bf16_bits(ze), jnp.int32(16)))


def _make_tables(symbol_emb, gcn_W, bias_row):
    n = symbol_emb.shape[0]
    blk = 2048
    grid = (n + blk - 1) // blk
    return pl.pallas_call(
        _tables_body,
        grid=(grid,),
        in_specs=[
            pl.BlockSpec((blk, D), lambda i: (i, 0)),
            pl.BlockSpec((D, 2 * D), lambda i: (0, 0)),
            pl.BlockSpec((1, D), lambda i: (0, 0)),
        ],
        out_specs=pl.BlockSpec((blk, D), lambda i: (i, 0)),
        out_shape=jax.ShapeDtypeStruct((n, D), jnp.int32),
    )(symbol_emb, gcn_W, bias_row)


def _sc_body(ptab_hbm, emb_hbm, rel_hbm, ent_hbm, self_hbm,
             agg_hbm, selfe_hbm,
             rel_idx, ent_idx, self_idx, rel_buf, ent_buf,
             out_stage, selfe_stage, sem_r, sem_e, sem_s):
    wid = lax.axis_index("s") * NC + lax.axis_index("c")

    pltpu.sync_copy(rel_hbm.at[pl.ds(wid * IDX_ROWS, IDX_ROWS)], rel_idx)
    pltpu.sync_copy(ent_hbm.at[pl.ds(wid * IDX_ROWS, IDX_ROWS)], ent_idx)
    pltpu.sync_copy(self_hbm.at[pl.ds(wid * 2, 2)], self_idx)

    for k in range(2):
        pltpu.async_copy(
            emb_hbm.at[self_idx.at[k]],
            selfe_stage.at[pl.ds(k * SELF_MINOR, SELF_MINOR)],
            sem_s,
        ).wait()
    pltpu.sync_copy(selfe_stage, selfe_hbm.at[pl.ds(wid * RPW, RPW)])

    sems = (sem_r, sem_e)

    def _issue(c, slot):
        pltpu.async_copy(ptab_hbm.at[rel_idx.at[c]], rel_buf.at[slot], sems[slot])
        pltpu.async_copy(ptab_hbm.at[ent_idx.at[c]], ent_buf.at[slot], sems[slot])

    def _wait(c, slot):
        pltpu.make_async_copy(ptab_hbm.at[rel_idx.at[c]], rel_buf.at[slot],
                              sems[slot]).wait()
        pltpu.make_async_copy(ptab_hbm.at[ent_idx.at[c]], ent_buf.at[slot],
                              sems[slot]).wait()

    def _compute(c, slot):
        himask = jnp.int32(-65536)

        for r in range(2):
            def nbody(n, acc):
                row = r * NB + n
                new = [None] * 8
                for g in range(8):
                    rw = rel_buf[slot, row, pl.ds(16 * g, 16)]
                    ew = ent_buf[slot, row, pl.ds(16 * g, 16)]
                    v = (lax.bitcast_convert_type(
                             lax.shift_left(rw, jnp.int32(16)), jnp.float32)
                         + lax.bitcast_convert_type(
                             jnp.bitwise_and(ew, himask), jnp.float32))
                    new[g] = acc[g] + jnp.maximum(v, v * 0.01)
                return tuple(new)
            acc0 = tuple(jnp.zeros((16,), jnp.float32) for _ in range(8))
            acc = lax.fori_loop(0, NB, nbody, acc0)
            for j in range(8):
                out_stage[2 * c + r, pl.ds(16 * j, 16)] = acc[j] * (1.0 / NB)

    _issue(0, 0)
    _issue(1, 1)

    def outer(c0, carry):
        for b in range(2):
            c = 2 * c0 + b
            _wait(c, b)
            _compute(c, b)
            nxt = c + 2

            @pl.when(nxt < IDX_ROWS)
            def _():
                _issue(nxt, b)
        return carry

    lax.fori_loop(0, IDX_ROWS // 2, outer, 0)
    pltpu.sync_copy(out_stage, agg_hbm.at[pl.ds(wid * RPW, RPW)])


def _sc_agg(ptab, emb, rel2d, ent2d, self2d):
    mesh = plsc.VectorSubcoreMesh(core_axis_name="c", subcore_axis_name="s")
    f = pl.kernel(
        _sc_body,
        mesh=mesh,
        out_type=(
            jax.ShapeDtypeStruct((ROWS, D), jnp.float32),
            jax.ShapeDtypeStruct((ROWS, D), jnp.float32),
        ),
        scratch_types=(
            pltpu.VMEM((IDX_ROWS, 128), jnp.int32),
            pltpu.VMEM((IDX_ROWS, 128), jnp.int32),
            pltpu.VMEM((2, SELF_MINOR), jnp.int32),
            pltpu.VMEM((2, 128, D), jnp.int32),
            pltpu.VMEM((2, 128, D), jnp.int32),
            pltpu.VMEM((RPW, D), jnp.float32),
            pltpu.VMEM((RPW, D), jnp.float32),
            pltpu.SemaphoreType.DMA,
            pltpu.SemaphoreType.DMA,
            pltpu.SemaphoreType.DMA,
        ),
    )
    return f(ptab, emb, rel2d, ent2d, self2d)


def _enc_body(agg, selfe, gate_w, gate_c, p1w, p1b, p2w, p2b, lna, lnb,
              q_enc_out, sg_out):
    a = agg[...]
    s = selfe[...]
    logit = jnp.sum(a * gate_w[...], axis=-1, keepdims=True) + gate_c[0, 0]
    gate = jax.nn.sigmoid(logit)
    fin = jnp.tanh(gate * a + (1.0 - gate) * s)

    qv = jnp.concatenate([fin[0:BQ], fin[BQ:2 * BQ]], axis=1)
    sv = jnp.concatenate([fin[2 * BQ:2 * BQ + BS], fin[2 * BQ + BS:]], axis=1)
    x = jnp.concatenate([qv, sv], axis=0)

    h1 = lax.dot_general(x.astype(jnp.bfloat16), p1w[...].astype(jnp.bfloat16),
                         (((1,), (1,)), ((), ())),
                         preferred_element_type=jnp.float32) + p1b[...]
    h1 = jnp.maximum(h1, 0.0)
    out = lax.dot_general(h1.astype(jnp.bfloat16), p2w[...].astype(jnp.bfloat16),
                          (((1,), (1,)), ((), ())),
                          preferred_element_type=jnp.float32) + p2b[...] + x
    mu = jnp.mean(out, axis=-1, keepdims=True)
    dlt = out - mu
    var = jnp.sum(dlt * dlt, axis=-1, keepdims=True) * (1.0 / (DM - 1))
    z = dlt / (jnp.sqrt(var) + 1e-6) * lna[...] + lnb[...]

    q_enc_out[...] = z[0:BQ]
    sg_out[...] = jnp.mean(z[BQ:], axis=0, keepdims=True)


def _encode(agg, selfe, gate_w, gate_c, p1w, p1b, p2w, p2b, lna, lnb):
    return pl.pallas_call(
        _enc_body,
        out_shape=[
            jax.ShapeDtypeStruct((BQ, DM), jnp.float32),
            jax.ShapeDtypeStruct((1, DM), jnp.float32),
        ],
    )(agg, selfe, gate_w, gate_c, p1w, p1b, p2w, p2b, lna, lnb)


def _lstm_body(q_enc, sg, wih, whA, whS, be, out):
    x = q_enc[...]
    sgv = sg[...]
    xb = x.astype(jnp.bfloat16)
    qW = lax.dot_general(xb, wih[...].astype(jnp.bfloat16),
                         (((1,), (1,)), ((), ())),
                         preferred_element_type=jnp.float32) + be[...]
    sS = lax.dot_general(sgv, whS[...], (((1,), (1,)), ((), ())),
                         preferred_element_type=jnp.float32)
    whAb = whA[...].astype(jnp.bfloat16)
    c = jnp.zeros((x.shape[0], DM), jnp.float32)
    h = x
    for step in range(4):
        if step == 0:
            gates = qW
        else:
            gates = qW + lax.dot_general(
                h.astype(jnp.bfloat16), whAb, (((1,), (1,)), ((), ())),
                preferred_element_type=jnp.float32) + sS
        i = jax.nn.sigmoid(gates[:, 0:DM])
        f = jax.nn.sigmoid(gates[:, DM:2 * DM])
        g = jnp.tanh(gates[:, 2 * DM:3 * DM])
        o = jax.nn.sigmoid(gates[:, 3 * DM:4 * DM])
        c = f * c + i * g
        h = x + o * jnp.tanh(c)
    scores = jnp.sum(h * sgv, axis=1)
    out[...] = scores.reshape(1, 1, -1)


def _lstm(q_enc, sg, wih, whA, whS, be):
    blk = 512
    nblk = BQ // blk
    out = pl.pallas_call(
        _lstm_body,
        grid=(nblk,),
        in_specs=[
            pl.BlockSpec((blk, DM), lambda i: (i, 0)),
            pl.BlockSpec((1, DM), lambda i: (0, 0)),
            pl.BlockSpec((4 * DM, DM), lambda i: (0, 0)),
            pl.BlockSpec((4 * DM, DM), lambda i: (0, 0)),
            pl.BlockSpec((4 * DM, DM), lambda i: (0, 0)),
            pl.BlockSpec((1, 4 * DM), lambda i: (0, 0)),
        ],
        out_specs=pl.BlockSpec((1, 1, blk), lambda i: (i, 0, 0)),
        out_shape=jax.ShapeDtypeStruct((nblk, 1, blk), jnp.float32),
    )(q_enc, sg, wih, whA, whS, be)
    return out.reshape(BQ)


def kernel(query, support, q_l1, q_l2, q_deg_l, q_r1, q_r2, q_deg_r,
           s_l1, s_l2, s_deg_l, s_r1, s_r2, s_deg_r, symbol_emb,
           gcn_W, gcn_bias, gcn_b, gate_W, gate_bias, gate_b,
           p1_W, p1_b, p2_W, p2_b, ln_a, ln_b, W_ih, W_hh, b_ih, b_hh):
    conn = jnp.concatenate([q_l1, q_r1, s_l1, s_r1], axis=0).astype(jnp.int32)
    rel2d = conn[:, :, 0].reshape(NW * IDX_ROWS, 128)
    ent2d = conn[:, :, 1].reshape(NW * IDX_ROWS, 128)
    self2d = jnp.concatenate(
        [query[:, 0], query[:, 1], support[:, 0], support[:, 1]]
    ).astype(jnp.int32).reshape(NW * 2, SELF_MINOR)

    bias_row = (gcn_bias + gcn_b).reshape(1, D)
    ptab = _make_tables(symbol_emb, gcn_W, bias_row)

    agg, selfe = _sc_agg(ptab, symbol_emb, rel2d, ent2d, self2d)

    gate_c = (gate_bias + gate_b).reshape(1, 1)
    q_enc, sg = _encode(agg, selfe, gate_W, gate_c,
                        p1_W, p1_b.reshape(1, -1), p2_W, p2_b.reshape(1, -1),
                        ln_a.reshape(1, -1), ln_b.reshape(1, -1))

    wih_e = jnp.concatenate([W_ih[0:DM], W_ih[H:H + DM],
                             W_ih[2 * H:2 * H + DM], W_ih[3 * H:3 * H + DM]], axis=0)
    whh_e = jnp.concatenate([W_hh[0:DM], W_hh[H:H + DM],
                             W_hh[2 * H:2 * H + DM], W_hh[3 * H:3 * H + DM]], axis=0)
    be = (b_ih + b_hh)
    be_e = jnp.concatenate([be[0:DM], be[H:H + DM],
                            be[2 * H:2 * H + DM], be[3 * H:3 * H + DM]]).reshape(1, -1)
    whA = whh_e[:, :DM]
    whS = whh_e[:, DM:]

    return _lstm(q_enc, sg, wih_e, whA, whS, be_e)


# speedup vs baseline: 1.0992x; 1.0992x over previous
import functools

import jax
import jax.numpy as jnp
import numpy as np
from jax import lax
from jax.experimental import pallas as pl
from jax.experimental.pallas import tpu as pltpu
from jax.experimental.pallas import tpu_sc as plsc

D = 128
DM = 256
H = 512
BQ = 2048
BS = 256
NB = 64
ROWS = 2 * BQ + 2 * BS

NC = 2
NS = 16
NW = NC * NS
RPW = ROWS // NW
IDX_ROWS = RPW * NB // 128
SELF_MINOR = RPW // 2


def _bf16_bits(x):
    h = lax.bitcast_convert_type(x.astype(jnp.bfloat16), jnp.int16)
    return jnp.bitwise_and(h.astype(jnp.int32), jnp.int32(0xFFFF))


def _bf16_bits(x):
    h = lax.bitcast_convert_type(x.astype(jnp.bfloat16), jnp.int16)
    return jnp.bitwise_and(h.astype(jnp.int32), jnp.int32(0xFFFF))


def _tables_body(emb, w, bias, ptab):
    e = emb[...].astype(jnp.bfloat16)
    w_all = w[...].astype(jnp.bfloat16)
    zr = lax.dot_general(e, w_all[:, :D], (((1,), (1,)), ((), ())),
                         preferred_element_type=jnp.float32)
    ze = lax.dot_general(e, w_all[:, D:], (((1,), (1,)), ((), ())),
                         preferred_element_type=jnp.float32) + bias[...]

    ptab[...] = jnp.bitwise_or(_bf16_bits(zr),
                               lax.shift_left(_bf16_bits(ze), jnp.int32(16)))


def _make_tables(symbol_emb, gcn_W, bias_row):
    n = symbol_emb.shape[0]
    blk = 8192
    grid = (n + blk - 1) // blk
    return pl.pallas_call(
        _tables_body,
        grid=(grid,),
        in_specs=[
            pl.BlockSpec((blk, D), lambda i: (i, 0)),
            pl.BlockSpec((D, 2 * D), lambda i: (0, 0)),
            pl.BlockSpec((1, D), lambda i: (0, 0)),
        ],
        out_specs=pl.BlockSpec((blk, D), lambda i: (i, 0)),
        out_shape=jax.ShapeDtypeStruct((n, D), jnp.int32),
    )(symbol_emb, gcn_W, bias_row)


def _sc_body(ptab_hbm, emb_hbm, rel_hbm, ent_hbm, self_hbm,
             agg_hbm, selfe_hbm,
             rel_idx, ent_idx, self_idx, rel_buf, ent_buf,
             out_stage, selfe_stage, sem_r, sem_e, sem_s):
    wid = lax.axis_index("s") * NC + lax.axis_index("c")

    pltpu.sync_copy(rel_hbm.at[pl.ds(wid * IDX_ROWS, IDX_ROWS)], rel_idx)
    pltpu.sync_copy(ent_hbm.at[pl.ds(wid * IDX_ROWS, IDX_ROWS)], ent_idx)
    pltpu.sync_copy(self_hbm.at[pl.ds(wid * 2, 2)], self_idx)

    for k in range(2):
        pltpu.async_copy(
            emb_hbm.at[self_idx.at[k]],
            selfe_stage.at[pl.ds(k * SELF_MINOR, SELF_MINOR)],
            sem_s,
        ).wait()
    pltpu.sync_copy(selfe_stage, selfe_hbm.at[pl.ds(wid * RPW, RPW)])

    sems = (sem_r, sem_e)

    def _issue(c, slot):
        pltpu.async_copy(ptab_hbm.at[rel_idx.at[c]], rel_buf.at[slot], sems[slot])
        pltpu.async_copy(ptab_hbm.at[ent_idx.at[c]], ent_buf.at[slot], sems[slot])

    def _wait(c, slot):
        pltpu.make_async_copy(ptab_hbm.at[rel_idx.at[c]], rel_buf.at[slot],
                              sems[slot]).wait()
        pltpu.make_async_copy(ptab_hbm.at[ent_idx.at[c]], ent_buf.at[slot],
                              sems[slot]).wait()

    def _compute(c, slot):
        himask = jnp.int32(-65536)

        for r in range(2):
            def nbody(n, acc):
                row = r * NB + n
                new = [None] * 8
                for g in range(8):
                    rw = rel_buf[slot, row, pl.ds(16 * g, 16)]
                    ew = ent_buf[slot, row, pl.ds(16 * g, 16)]
                    v = (lax.bitcast_convert_type(
                             lax.shift_left(rw, jnp.int32(16)), jnp.float32)
                         + lax.bitcast_convert_type(
                             jnp.bitwise_and(ew, himask), jnp.float32))
                    new[g] = acc[g] + jnp.maximum(v, v * 0.01)
                return tuple(new)
            acc0 = tuple(jnp.zeros((16,), jnp.float32) for _ in range(8))
            acc = lax.fori_loop(0, NB, nbody, acc0)
            for j in range(8):
                out_stage[2 * c + r, pl.ds(16 * j, 16)] = acc[j] * (1.0 / NB)

    _issue(0, 0)
    _issue(1, 1)

    def outer(c0, carry):
        for b in range(2):
            c = 2 * c0 + b
            _wait(c, b)
            _compute(c, b)
            nxt = c + 2

            @pl.when(nxt < IDX_ROWS)
            def _():
                _issue(nxt, b)
        return carry

    lax.fori_loop(0, IDX_ROWS // 2, outer, 0)
    pltpu.sync_copy(out_stage, agg_hbm.at[pl.ds(wid * RPW, RPW)])


def _sc_agg(ptab, emb, rel2d, ent2d, self2d):
    mesh = plsc.VectorSubcoreMesh(core_axis_name="c", subcore_axis_name="s")
    f = pl.kernel(
        _sc_body,
        mesh=mesh,
        out_type=(
            jax.ShapeDtypeStruct((ROWS, D), jnp.float32),
            jax.ShapeDtypeStruct((ROWS, D), jnp.float32),
        ),
        scratch_types=(
            pltpu.VMEM((IDX_ROWS, 128), jnp.int32),
            pltpu.VMEM((IDX_ROWS, 128), jnp.int32),
            pltpu.VMEM((2, SELF_MINOR), jnp.int32),
            pltpu.VMEM((2, 128, D), jnp.int32),
            pltpu.VMEM((2, 128, D), jnp.int32),
            pltpu.VMEM((RPW, D), jnp.float32),
            pltpu.VMEM((RPW, D), jnp.float32),
            pltpu.SemaphoreType.DMA,
            pltpu.SemaphoreType.DMA,
            pltpu.SemaphoreType.DMA,
        ),
    )
    return f(ptab, emb, rel2d, ent2d, self2d)


def _enc_body(agg, selfe, gate_w, gate_c, p1w, p1b, p2w, p2b, lna, lnb,
              q_enc_out, sg_out):
    a = agg[...]
    s = selfe[...]
    logit = jnp.sum(a * gate_w[...], axis=-1, keepdims=True) + gate_c[0, 0]
    gate = jax.nn.sigmoid(logit)
    fin = jnp.tanh(gate * a + (1.0 - gate) * s)

    qv = jnp.concatenate([fin[0:BQ], fin[BQ:2 * BQ]], axis=1)
    sv = jnp.concatenate([fin[2 * BQ:2 * BQ + BS], fin[2 * BQ + BS:]], axis=1)
    x = jnp.concatenate([qv, sv], axis=0)

    h1 = lax.dot_general(x.astype(jnp.bfloat16), p1w[...].astype(jnp.bfloat16),
                         (((1,), (1,)), ((), ())),
                         preferred_element_type=jnp.float32) + p1b[...]
    h1 = jnp.maximum(h1, 0.0)
    out = lax.dot_general(h1.astype(jnp.bfloat16), p2w[...].astype(jnp.bfloat16),
                          (((1,), (1,)), ((), ())),
                          preferred_element_type=jnp.float32) + p2b[...] + x
    mu = jnp.mean(out, axis=-1, keepdims=True)
    dlt = out - mu
    var = jnp.sum(dlt * dlt, axis=-1, keepdims=True) * (1.0 / (DM - 1))
    z = dlt / (jnp.sqrt(var) + 1e-6) * lna[...] + lnb[...]

    q_enc_out[...] = z[0:BQ]
    sg_out[...] = jnp.mean(z[BQ:], axis=0, keepdims=True)


def _encode(agg, selfe, gate_w, gate_c, p1w, p1b, p2w, p2b, lna, lnb):
    return pl.pallas_call(
        _enc_body,
        out_shape=[
            jax.ShapeDtypeStruct((BQ, DM), jnp.float32),
            jax.ShapeDtypeStruct((1, DM), jnp.float32),
        ],
    )(agg, selfe, gate_w, gate_c, p1w, p1b, p2w, p2b, lna, lnb)


def _lstm_body(q_enc, sg, wih, whA, whS, be, out):
    x = q_enc[...]
    sgv = sg[...]
    xb = x.astype(jnp.bfloat16)
    qW = lax.dot_general(xb, wih[...], (((1,), (1,)), ((), ())),
                         preferred_element_type=jnp.float32) + be[...]
    sS = lax.dot_general(sgv, whS[...], (((1,), (1,)), ((), ())),
                         preferred_element_type=jnp.float32)
    whAb = whA[...]
    c = jnp.zeros((x.shape[0], DM), jnp.float32)
    h = x
    for step in range(4):
        if step == 0:
            gates = qW
        else:
            gates = qW + lax.dot_general(
                h.astype(jnp.bfloat16), whAb, (((1,), (1,)), ((), ())),
                preferred_element_type=jnp.float32) + sS
        i = jax.nn.sigmoid(gates[:, 0:DM])
        f = jax.nn.sigmoid(gates[:, DM:2 * DM])
        g = jnp.tanh(gates[:, 2 * DM:3 * DM])
        o = jax.nn.sigmoid(gates[:, 3 * DM:4 * DM])
        c = f * c + i * g
        h = x + o * jnp.tanh(c)
    scores = jnp.sum(h * sgv, axis=1)
    out[...] = scores.reshape(1, 1, -1)


def _lstm(q_enc, sg, wih, whA, whS, be):
    blk = BQ
    nblk = BQ // blk
    out = pl.pallas_call(
        _lstm_body,
        grid=(nblk,),
        in_specs=[
            pl.BlockSpec((blk, DM), lambda i: (i, 0)),
            pl.BlockSpec((1, DM), lambda i: (0, 0)),
            pl.BlockSpec((4 * DM, DM), lambda i: (0, 0)),
            pl.BlockSpec((4 * DM, DM), lambda i: (0, 0)),
            pl.BlockSpec((4 * DM, DM), lambda i: (0, 0)),
            pl.BlockSpec((1, 4 * DM), lambda i: (0, 0)),
        ],
        out_specs=pl.BlockSpec((1, 1, blk), lambda i: (i, 0, 0)),
        out_shape=jax.ShapeDtypeStruct((nblk, 1, blk), jnp.float32),
    )(q_enc, sg, wih, whA, whS, be)
    return out.reshape(BQ)


def kernel(query, support, q_l1, q_l2, q_deg_l, q_r1, q_r2, q_deg_r,
           s_l1, s_l2, s_deg_l, s_r1, s_r2, s_deg_r, symbol_emb,
           gcn_W, gcn_bias, gcn_b, gate_W, gate_bias, gate_b,
           p1_W, p1_b, p2_W, p2_b, ln_a, ln_b, W_ih, W_hh, b_ih, b_hh):
    conn = jnp.concatenate([q_l1, q_r1, s_l1, s_r1], axis=0).astype(jnp.int32)
    rel2d = conn[:, :, 0].reshape(NW * IDX_ROWS, 128)
    ent2d = conn[:, :, 1].reshape(NW * IDX_ROWS, 128)
    self2d = jnp.concatenate(
        [query[:, 0], query[:, 1], support[:, 0], support[:, 1]]
    ).astype(jnp.int32).reshape(NW * 2, SELF_MINOR)

    bias_row = (gcn_bias + gcn_b).reshape(1, D)
    ptab = _make_tables(symbol_emb, gcn_W, bias_row)

    agg, selfe = _sc_agg(ptab, symbol_emb, rel2d, ent2d, self2d)

    gate_c = (gate_bias + gate_b).reshape(1, 1)
    q_enc, sg = _encode(agg, selfe, gate_W, gate_c,
                        p1_W, p1_b.reshape(1, -1), p2_W, p2_b.reshape(1, -1),
                        ln_a.reshape(1, -1), ln_b.reshape(1, -1))

    wih_e = jnp.concatenate([W_ih[0:DM], W_ih[H:H + DM],
                             W_ih[2 * H:2 * H + DM], W_ih[3 * H:3 * H + DM]], axis=0)
    whh_e = jnp.concatenate([W_hh[0:DM], W_hh[H:H + DM],
                             W_hh[2 * H:2 * H + DM], W_hh[3 * H:3 * H + DM]], axis=0)
    be = (b_ih + b_hh)
    be_e = jnp.concatenate([be[0:DM], be[H:H + DM],
                            be[2 * H:2 * H + DM], be[3 * H:3 * H + DM]]).reshape(1, -1)
    whA = whh_e[:, :DM].astype(jnp.bfloat16)
    whS = whh_e[:, DM:]

    return _lstm(q_enc, sg, wih_e.astype(jnp.bfloat16), whA, whS, be_e)


# speedup vs baseline: 1.1174x; 1.0165x over previous
import functools

import jax
import jax.numpy as jnp
import numpy as np
from jax import lax
from jax.experimental import pallas as pl
from jax.experimental.pallas import tpu as pltpu
from jax.experimental.pallas import tpu_sc as plsc

D = 128
DM = 256
H = 512
BQ = 2048
BS = 256
NB = 64
ROWS = 2 * BQ + 2 * BS

NC = 2
NS = 16
NW = NC * NS
RPW = ROWS // NW
IDX_ROWS = RPW * NB // 128
SELF_MINOR = RPW // 2


def _bf16_bits(x):
    h = lax.bitcast_convert_type(x.astype(jnp.bfloat16), jnp.int16)
    return jnp.bitwise_and(h.astype(jnp.int32), jnp.int32(0xFFFF))


def _bf16_bits(x):
    h = lax.bitcast_convert_type(x.astype(jnp.bfloat16), jnp.int16)
    return jnp.bitwise_and(h.astype(jnp.int32), jnp.int32(0xFFFF))


def _tables_body(emb, w, bias, ptab):
    e = emb[...].astype(jnp.bfloat16)
    w_all = w[...].astype(jnp.bfloat16)
    zr = lax.dot_general(e, w_all[:, :D], (((1,), (1,)), ((), ())),
                         preferred_element_type=jnp.float32)
    ze = lax.dot_general(e, w_all[:, D:], (((1,), (1,)), ((), ())),
                         preferred_element_type=jnp.float32) + bias[...]

    ptab[...] = jnp.bitwise_or(_bf16_bits(zr),
                               lax.shift_left(_bf16_bits(ze), jnp.int32(16)))


def _make_tables(symbol_emb, gcn_W, bias_row):
    n = symbol_emb.shape[0]
    blk = 16384
    grid = (n + blk - 1) // blk
    return pl.pallas_call(
        _tables_body,
        grid=(grid,),
        in_specs=[
            pl.BlockSpec((blk, D), lambda i: (i, 0)),
            pl.BlockSpec((D, 2 * D), lambda i: (0, 0)),
            pl.BlockSpec((1, D), lambda i: (0, 0)),
        ],
        out_specs=pl.BlockSpec((blk, D), lambda i: (i, 0)),
        out_shape=jax.ShapeDtypeStruct((n, D), jnp.int32),
    )(symbol_emb, gcn_W, bias_row)


def _sc_body(ptab_hbm, emb_hbm, rel_hbm, ent_hbm, self_hbm,
             agg_hbm, selfe_hbm,
             rel_idx, ent_idx, self_idx, rel_buf, ent_buf,
             out_stage, selfe_stage, sem_r, sem_e, sem_s):
    wid = lax.axis_index("s") * NC + lax.axis_index("c")

    ra = pltpu.async_copy(rel_hbm.at[pl.ds(wid * IDX_ROWS, IDX_ROWS)],
                          rel_idx, sem_r)
    ea = pltpu.async_copy(ent_hbm.at[pl.ds(wid * IDX_ROWS, IDX_ROWS)],
                          ent_idx, sem_e)
    sa = pltpu.async_copy(self_hbm.at[pl.ds(wid * 2, 2)], self_idx, sem_s)
    ra.wait()
    ea.wait()

    sems = (sem_r, sem_e)

    def _issue(c, slot):
        pltpu.async_copy(ptab_hbm.at[rel_idx.at[c]], rel_buf.at[slot], sems[slot])
        pltpu.async_copy(ptab_hbm.at[ent_idx.at[c]], ent_buf.at[slot], sems[slot])

    def _wait(c, slot):
        pltpu.make_async_copy(ptab_hbm.at[rel_idx.at[c]], rel_buf.at[slot],
                              sems[slot]).wait()
        pltpu.make_async_copy(ptab_hbm.at[ent_idx.at[c]], ent_buf.at[slot],
                              sems[slot]).wait()

    def _compute(c, slot):
        himask = jnp.int32(-65536)

        for r in range(2):
            def nbody(n, acc):
                row = r * NB + n
                new = [None] * 8
                for g in range(8):
                    rw = rel_buf[slot, row, pl.ds(16 * g, 16)]
                    ew = ent_buf[slot, row, pl.ds(16 * g, 16)]
                    v = (lax.bitcast_convert_type(
                             lax.shift_left(rw, jnp.int32(16)), jnp.float32)
                         + lax.bitcast_convert_type(
                             jnp.bitwise_and(ew, himask), jnp.float32))
                    new[g] = acc[g] + jnp.maximum(v, v * 0.01)
                return tuple(new)
            acc0 = tuple(jnp.zeros((16,), jnp.float32) for _ in range(8))
            acc = lax.fori_loop(0, NB, nbody, acc0)
            for j in range(8):
                out_stage[2 * c + r, pl.ds(16 * j, 16)] = acc[j] * (1.0 / NB)

    _issue(0, 0)
    _issue(1, 1)

    sa.wait()
    for k in range(2):
        pltpu.async_copy(
            emb_hbm.at[self_idx.at[k]],
            selfe_stage.at[pl.ds(k * SELF_MINOR, SELF_MINOR)],
            sem_s,
        ).wait()
    pltpu.sync_copy(selfe_stage, selfe_hbm.at[pl.ds(wid * RPW, RPW)])

    def outer(c0, carry):
        for b in range(2):
            c = 2 * c0 + b
            _wait(c, b)
            _compute(c, b)
            nxt = c + 2

            @pl.when(nxt < IDX_ROWS)
            def _():
                _issue(nxt, b)
        return carry

    lax.fori_loop(0, IDX_ROWS // 2, outer, 0)
    pltpu.sync_copy(out_stage, agg_hbm.at[pl.ds(wid * RPW, RPW)])


def _sc_agg(ptab, emb, rel2d, ent2d, self2d):
    mesh = plsc.VectorSubcoreMesh(core_axis_name="c", subcore_axis_name="s")
    f = pl.kernel(
        _sc_body,
        mesh=mesh,
        out_type=(
            jax.ShapeDtypeStruct((ROWS, D), jnp.float32),
            jax.ShapeDtypeStruct((ROWS, D), jnp.float32),
        ),
        scratch_types=(
            pltpu.VMEM((IDX_ROWS, 128), jnp.int32),
            pltpu.VMEM((IDX_ROWS, 128), jnp.int32),
            pltpu.VMEM((2, SELF_MINOR), jnp.int32),
            pltpu.VMEM((2, 128, D), jnp.int32),
            pltpu.VMEM((2, 128, D), jnp.int32),
            pltpu.VMEM((RPW, D), jnp.float32),
            pltpu.VMEM((RPW, D), jnp.float32),
            pltpu.SemaphoreType.DMA,
            pltpu.SemaphoreType.DMA,
            pltpu.SemaphoreType.DMA,
        ),
    )
    return f(ptab, emb, rel2d, ent2d, self2d)


def _enc_body(agg, selfe, gate_w, gate_c, p1w, p1b, p2w, p2b, lna, lnb,
              q_enc_out, sg_out):
    a = agg[...]
    s = selfe[...]
    logit = jnp.sum(a * gate_w[...], axis=-1, keepdims=True) + gate_c[0, 0]
    gate = jax.nn.sigmoid(logit)
    fin = jnp.tanh(gate * a + (1.0 - gate) * s)

    qv = jnp.concatenate([fin[0:BQ], fin[BQ:2 * BQ]], axis=1)
    sv = jnp.concatenate([fin[2 * BQ:2 * BQ + BS], fin[2 * BQ + BS:]], axis=1)
    x = jnp.concatenate([qv, sv], axis=0)

    h1 = lax.dot_general(x.astype(jnp.bfloat16), p1w[...].astype(jnp.bfloat16),
                         (((1,), (1,)), ((), ())),
                         preferred_element_type=jnp.float32) + p1b[...]
    h1 = jnp.maximum(h1, 0.0)
    out = lax.dot_general(h1.astype(jnp.bfloat16), p2w[...].astype(jnp.bfloat16),
                          (((1,), (1,)), ((), ())),
                          preferred_element_type=jnp.float32) + p2b[...] + x
    mu = jnp.mean(out, axis=-1, keepdims=True)
    dlt = out - mu
    var = jnp.sum(dlt * dlt, axis=-1, keepdims=True) * (1.0 / (DM - 1))
    z = dlt / (jnp.sqrt(var) + 1e-6) * lna[...] + lnb[...]

    q_enc_out[...] = z[0:BQ]
    sg_out[...] = jnp.mean(z[BQ:], axis=0, keepdims=True)


def _encode(agg, selfe, gate_w, gate_c, p1w, p1b, p2w, p2b, lna, lnb):
    return pl.pallas_call(
        _enc_body,
        out_shape=[
            jax.ShapeDtypeStruct((BQ, DM), jnp.float32),
            jax.ShapeDtypeStruct((1, DM), jnp.float32),
        ],
    )(agg, selfe, gate_w, gate_c, p1w, p1b, p2w, p2b, lna, lnb)


def _lstm_body(q_enc, sg, wih, whA, whS, be, out):
    x = q_enc[...]
    sgv = sg[...]
    xb = x.astype(jnp.bfloat16)
    qW = lax.dot_general(xb, wih[...], (((1,), (1,)), ((), ())),
                         preferred_element_type=jnp.float32) + be[...]
    sS = lax.dot_general(sgv, whS[...], (((1,), (1,)), ((), ())),
                         preferred_element_type=jnp.float32)
    whAb = whA[...]
    c = jnp.zeros((x.shape[0], DM), jnp.float32)
    h = x
    for step in range(4):
        if step == 0:
            gates = qW
        else:
            gates = qW + lax.dot_general(
                h.astype(jnp.bfloat16), whAb, (((1,), (1,)), ((), ())),
                preferred_element_type=jnp.float32) + sS
        i = jax.nn.sigmoid(gates[:, 0:DM])
        f = jax.nn.sigmoid(gates[:, DM:2 * DM])
        g = jnp.tanh(gates[:, 2 * DM:3 * DM])
        o = jax.nn.sigmoid(gates[:, 3 * DM:4 * DM])
        c = f * c + i * g
        h = x + o * jnp.tanh(c)
    scores = jnp.sum(h * sgv, axis=1)
    out[...] = scores.reshape(1, 1, -1)


def _lstm(q_enc, sg, wih, whA, whS, be):
    blk = BQ
    nblk = BQ // blk
    out = pl.pallas_call(
        _lstm_body,
        grid=(nblk,),
        in_specs=[
            pl.BlockSpec((blk, DM), lambda i: (i, 0)),
            pl.BlockSpec((1, DM), lambda i: (0, 0)),
            pl.BlockSpec((4 * DM, DM), lambda i: (0, 0)),
            pl.BlockSpec((4 * DM, DM), lambda i: (0, 0)),
            pl.BlockSpec((4 * DM, DM), lambda i: (0, 0)),
            pl.BlockSpec((1, 4 * DM), lambda i: (0, 0)),
        ],
        out_specs=pl.BlockSpec((1, 1, blk), lambda i: (i, 0, 0)),
        out_shape=jax.ShapeDtypeStruct((nblk, 1, blk), jnp.float32),
    )(q_enc, sg, wih, whA, whS, be)
    return out.reshape(BQ)


def kernel(query, support, q_l1, q_l2, q_deg_l, q_r1, q_r2, q_deg_r,
           s_l1, s_l2, s_deg_l, s_r1, s_r2, s_deg_r, symbol_emb,
           gcn_W, gcn_bias, gcn_b, gate_W, gate_bias, gate_b,
           p1_W, p1_b, p2_W, p2_b, ln_a, ln_b, W_ih, W_hh, b_ih, b_hh):
    conn = jnp.concatenate([q_l1, q_r1, s_l1, s_r1], axis=0).astype(jnp.int32)
    rel2d = conn[:, :, 0].reshape(NW * IDX_ROWS, 128)
    ent2d = conn[:, :, 1].reshape(NW * IDX_ROWS, 128)
    self2d = jnp.concatenate(
        [query[:, 0], query[:, 1], support[:, 0], support[:, 1]]
    ).astype(jnp.int32).reshape(NW * 2, SELF_MINOR)

    bias_row = (gcn_bias + gcn_b).reshape(1, D)
    ptab = _make_tables(symbol_emb, gcn_W, bias_row)

    agg, selfe = _sc_agg(ptab, symbol_emb, rel2d, ent2d, self2d)

    gate_c = (gate_bias + gate_b).reshape(1, 1)
    q_enc, sg = _encode(agg, selfe, gate_W, gate_c,
                        p1_W, p1_b.reshape(1, -1), p2_W, p2_b.reshape(1, -1),
                        ln_a.reshape(1, -1), ln_b.reshape(1, -1))

    wih_e = jnp.concatenate([W_ih[0:DM], W_ih[H:H + DM],
                             W_ih[2 * H:2 * H + DM], W_ih[3 * H:3 * H + DM]], axis=0)
    whh_e = jnp.concatenate([W_hh[0:DM], W_hh[H:H + DM],
                             W_hh[2 * H:2 * H + DM], W_hh[3 * H:3 * H + DM]], axis=0)
    be = (b_ih + b_hh)
    be_e = jnp.concatenate([be[0:DM], be[H:H + DM],
                            be[2 * H:2 * H + DM], be[3 * H:3 * H + DM]]).reshape(1, -1)
    whA = whh_e[:, :DM].astype(jnp.bfloat16)
    whS = whh_e[:, DM:]

    return _lstm(q_enc, sg, wih_e.astype(jnp.bfloat16), whA, whS, be_e)
